# Initial kernel scaffold; baseline (speedup 1.0000x reference)
import functools

import jax
import jax.numpy as jnp
from jax import lax
from jax.experimental import pallas as pl
from jax.experimental.pallas import tpu as pltpu
from jax.experimental.pallas import tpu_sc as plsc

N = 10000
E = 160000
F = 16
NC = 2
NS = 16
NW = NC * NS
CH = 128
NCH = 5120 // CH
EP = NW * NCH * CH
NP = 10016
RPT = NP // NS

_f32 = jnp.float32


def _fill_rows(ref, n, val):
    @pl.loop(0, n)
    def _(i):
        ref[i, :] = jnp.full((16,), val, _f32)


def _sc_mesh():
    return plsc.VectorSubcoreMesh(core_axis_name="c", subcore_axis_name="s")


def _sc_degree(dst_flat):
    @functools.partial(
        pl.kernel,
        out_type=jax.ShapeDtypeStruct((NC, NP, F), _f32),
        mesh=_sc_mesh(),
        scratch_types=[
            pltpu.VMEM_SHARED((NP, F), _f32),
            pltpu.VMEM((CH,), jnp.int32),
            pltpu.VMEM((CH, F), _f32),
            pltpu.VMEM((RPT, F), _f32),
        ],
    )
    def k(dst_hbm, out_hbm, acc_sh, dstb, ones_b, stage):
        c = lax.axis_index("c")
        s = lax.axis_index("s")
        w = c * NS + s
        rows = pl.ds(s * RPT, RPT)

        _fill_rows(stage, RPT, 0.0)
        pltpu.sync_copy(stage, acc_sh.at[rows])
        _fill_rows(ones_b, CH, 1.0)
        plsc.subcore_barrier()

        @pl.loop(0, NCH)
        def _(jj):
            base = (w * NCH + jj) * CH
            pltpu.sync_copy(dst_hbm.at[pl.ds(base, CH)], dstb)
            pltpu.sync_copy(ones_b, acc_sh.at[dstb], add=True)

        plsc.subcore_barrier()
        pltpu.sync_copy(acc_sh.at[rows], stage)
        pltpu.sync_copy(stage, out_hbm.at[c, rows])

    return k(dst_flat)


def _sc_prop(table, src_flat, dst_flat):
    @functools.partial(
        pl.kernel,
        out_type=jax.ShapeDtypeStruct((NC, NP, F), _f32),
        mesh=_sc_mesh(),
        scratch_types=[
            pltpu.VMEM_SHARED((NP, F), _f32),
            pltpu.VMEM((NCH * CH,), jnp.int32),
            pltpu.VMEM((CH,), jnp.int32),
            pltpu.VMEM((CH, F), _f32),
            pltpu.VMEM((RPT, F), _f32),
            pltpu.SemaphoreType.DMA,
        ],
    )
    def k(tab_hbm, src_hbm, dst_hbm, out_hbm,
          acc_sh, srcb, dstb, rows_b, stage, sem):
        c = lax.axis_index("c")
        s = lax.axis_index("s")
        w = c * NS + s
        rows = pl.ds(s * RPT, RPT)

        _fill_rows(stage, RPT, 0.0)
        pltpu.sync_copy(stage, acc_sh.at[rows])
        pltpu.sync_copy(src_hbm.at[pl.ds(w * NCH * CH, NCH * CH)], srcb)
        plsc.subcore_barrier()

        @pl.loop(0, NCH)
        def _(jj):
            base = (w * NCH + jj) * CH
            pltpu.sync_copy(dst_hbm.at[pl.ds(base, CH)], dstb)
            pltpu.async_copy(
                tab_hbm.at[srcb.at[pl.ds(jj * CH, CH)]], rows_b, sem
            ).wait()
            pltpu.sync_copy(rows_b, acc_sh.at[dstb], add=True)

        plsc.subcore_barrier()
        pltpu.sync_copy(acc_sh.at[rows], stage)
        pltpu.sync_copy(stage, out_hbm.at[c, rows])

    return k(table, src_flat, dst_flat)


_BM = 1000


def _tc_first(x, W1, degp):
    def body(x_ref, w_ref, d_ref, t_ref, q_ref):
        dp = d_ref[0] + d_ref[1] + 1.0
        q = lax.rsqrt(dp)
        t = jnp.dot(x_ref[...], w_ref[...], preferred_element_type=_f32)
        t_ref[...] = q * t
        q_ref[...] = q

    return pl.pallas_call(
        body,
        grid=(N // _BM,),
        in_specs=[
            pl.BlockSpec((_BM, 640), lambda i: (i, 0)),
            pl.BlockSpec((640, F), lambda i: (0, 0)),
            pl.BlockSpec((NC, _BM, F), lambda i: (0, i, 0)),
        ],
        out_specs=[
            pl.BlockSpec((_BM, F), lambda i: (i, 0)),
            pl.BlockSpec((_BM, F), lambda i: (i, 0)),
        ],
        out_shape=[
            jax.ShapeDtypeStruct((N, F), _f32),
            jax.ShapeDtypeStruct((N, F), _f32),
        ],
    )(x, W1, degp)


def _tc_mid1(zp, t1s, q16, b1, W2):
    def body(z_ref, t_ref, q_ref, b_ref, w_ref, o_ref):
        q = q_ref[...]
        z = z_ref[0] + z_ref[1] + t_ref[...]
        a = jnp.maximum(q * z + b_ref[...], 0.0)
        o_ref[...] = jnp.dot(q * a, w_ref[...], preferred_element_type=_f32)

    return pl.pallas_call(
        body,
        grid=(N // _BM,),
        in_specs=[
            pl.BlockSpec((NC, _BM, F), lambda i: (0, i, 0)),
            pl.BlockSpec((_BM, F), lambda i: (i, 0)),
            pl.BlockSpec((_BM, F), lambda i: (i, 0)),
            pl.BlockSpec((1, F), lambda i: (0, 0)),
            pl.BlockSpec((F, F), lambda i: (0, 0)),
        ],
        out_specs=pl.BlockSpec((_BM, F), lambda i: (i, 0)),
        out_shape=jax.ShapeDtypeStruct((N, F), _f32),
    )(zp, t1s, q16, b1, W2)


def _tc_mid2(zp, t2s, q16, b2):
    def body(z_ref, t_ref, q_ref, b_ref, o_ref):
        q = q_ref[...]
        z = z_ref[0] + z_ref[1] + t_ref[...]
        o_ref[...] = q * jnp.maximum(q * z + b_ref[...], 0.0)

    return pl.pallas_call(
        body,
        grid=(N // _BM,),
        in_specs=[
            pl.BlockSpec((NC, _BM, F), lambda i: (0, i, 0)),
            pl.BlockSpec((_BM, F), lambda i: (i, 0)),
            pl.BlockSpec((_BM, F), lambda i: (i, 0)),
            pl.BlockSpec((1, F), lambda i: (0, 0)),
        ],
        out_specs=pl.BlockSpec((_BM, F), lambda i: (i, 0)),
        out_shape=jax.ShapeDtypeStruct((N, F), _f32),
    )(zp, t2s, q16, b2)


def _tc_last(zp, s3, q16, W3, b3):
    def body(z_ref, s_ref, q_ref, w_ref, b_ref, o_ref):
        p = q_ref[...] * (z_ref[0] + z_ref[1] + s_ref[...])
        o_ref[...] = (
            jnp.dot(p, w_ref[...], preferred_element_type=_f32) + b_ref[...]
        )

    return pl.pallas_call(
        body,
        grid=(N // _BM,),
        in_specs=[
            pl.BlockSpec((NC, _BM, F), lambda i: (0, i, 0)),
            pl.BlockSpec((_BM, F), lambda i: (i, 0)),
            pl.BlockSpec((_BM, F), lambda i: (i, 0)),
            pl.BlockSpec((F, 640), lambda i: (0, 0)),
            pl.BlockSpec((1, 640), lambda i: (0, 0)),
        ],
        out_specs=pl.BlockSpec((_BM, 640), lambda i: (i, 0)),
        out_shape=jax.ShapeDtypeStruct((N, 640), _f32),
    )(zp, s3, q16, W3, b3)


def _pad_table(t):
    return jnp.pad(t, ((0, NP - N), (0, 0)))


def kernel(x, edges, W1, b1, W2, b2, W3, b3):
    src = edges[:, 0]
    dst = edges[:, 1]
    dummy = (N + (jnp.arange(EP - E, dtype=jnp.int32) % NS)).astype(jnp.int32)
    src_p = jnp.concatenate([src, dummy])
    dst_p = jnp.concatenate([dst, dummy])

    b1r = b1.reshape(1, F)
    b2r = b2.reshape(1, F)
    b3r = b3.reshape(1, 640)

    degp = _sc_degree(dst_p)[:, :N]
    t1s, q16 = _tc_first(x, W1, degp)

    z1p = _sc_prop(_pad_table(t1s), src_p, dst_p)[:, :N]
    t2s = _tc_mid1(z1p, t1s, q16, b1r, W2)

    z2p = _sc_prop(_pad_table(t2s), src_p, dst_p)[:, :N]
    s3 = _tc_mid2(z2p, t2s, q16, b2r)

    z3p = _sc_prop(_pad_table(s3), src_p, dst_p)[:, :N]
    out = _tc_last(z3p, s3, q16, W3, b3r)
    return out


# speedup vs baseline: 22.4220x; 22.4220x over previous
import functools

import jax
import jax.numpy as jnp
from jax import lax
from jax.experimental import pallas as pl
from jax.experimental.pallas import tpu as pltpu
from jax.experimental.pallas import tpu_sc as plsc

N = 10000
E = 160000
F = 16
NC = 2
NS = 16
NW = NC * NS
CH = 128
NCH = 5120 // CH
EP = NW * NCH * CH
NP = 10112
RPT = NP // NS

_f32 = jnp.float32


def _fill_rows(ref, n, val):
    @pl.loop(0, n)
    def _(i):
        ref[i, :] = jnp.full((16,), val, _f32)


def _sc_mesh():
    return plsc.VectorSubcoreMesh(core_axis_name="c", subcore_axis_name="s")


_SC_PARAMS = pltpu.CompilerParams(use_tc_tiling_on_sc=False)


def _sc_degree(dst_flat):
    @functools.partial(
        pl.kernel,
        out_type=jax.ShapeDtypeStruct((NC, NP, F), _f32),
        mesh=_sc_mesh(),
        compiler_params=_SC_PARAMS,
        scratch_types=[
            pltpu.VMEM_SHARED((NP, F), _f32),
            pltpu.VMEM((CH,), jnp.int32),
            pltpu.VMEM((CH, F), _f32),
            pltpu.VMEM((RPT, F), _f32),
        ],
    )
    def k(dst_hbm, out_hbm, acc_sh, dstb, ones_b, stage):
        c = lax.axis_index("c")
        s = lax.axis_index("s")
        w = c * NS + s
        rows = pl.ds(s * RPT, RPT)

        _fill_rows(stage, RPT, 0.0)
        pltpu.sync_copy(stage, acc_sh.at[rows])
        _fill_rows(ones_b, CH, 1.0)
        plsc.subcore_barrier()

        @pl.loop(0, NCH)
        def _(jj):
            base = (w * NCH + jj) * CH
            pltpu.sync_copy(dst_hbm.at[pl.ds(base, CH)], dstb)
            pltpu.sync_copy(ones_b, acc_sh.at[dstb], add=True)

        plsc.subcore_barrier()
        pltpu.sync_copy(acc_sh.at[rows], stage)
        pltpu.sync_copy(stage, out_hbm.at[c, rows])

    return k(dst_flat)


def _sc_prop(table, src_flat, dst_flat):
    @functools.partial(
        pl.kernel,
        out_type=jax.ShapeDtypeStruct((NC, NP, F), _f32),
        mesh=_sc_mesh(),
        compiler_params=_SC_PARAMS,
        scratch_types=[
            pltpu.VMEM_SHARED((NP, F), _f32),
            pltpu.VMEM((NCH * CH,), jnp.int32),
            pltpu.VMEM((CH,), jnp.int32),
            pltpu.VMEM((CH, F), _f32),
            pltpu.VMEM((RPT, F), _f32),
            pltpu.SemaphoreType.DMA,
        ],
    )
    def k(tab_hbm, src_hbm, dst_hbm, out_hbm,
          acc_sh, srcb, dstb, rows_b, stage, sem):
        c = lax.axis_index("c")
        s = lax.axis_index("s")
        w = c * NS + s
        rows = pl.ds(s * RPT, RPT)

        _fill_rows(stage, RPT, 0.0)
        pltpu.sync_copy(stage, acc_sh.at[rows])
        pltpu.sync_copy(src_hbm.at[pl.ds(w * NCH * CH, NCH * CH)], srcb)
        plsc.subcore_barrier()

        @pl.loop(0, NCH)
        def _(jj):
            base = (w * NCH + jj) * CH
            pltpu.sync_copy(dst_hbm.at[pl.ds(base, CH)], dstb)
            pltpu.async_copy(
                tab_hbm.at[srcb.at[pl.ds(jj * CH, CH)]], rows_b, sem
            ).wait()
            pltpu.sync_copy(rows_b, acc_sh.at[dstb], add=True)

        plsc.subcore_barrier()
        pltpu.sync_copy(acc_sh.at[rows], stage)
        pltpu.sync_copy(stage, out_hbm.at[c, rows])

    return k(table, src_flat, dst_flat)


_BM = 1000


def _tc_first(x, W1, degp):
    def body(x_ref, w_ref, d_ref, t_ref, q_ref):
        dp = d_ref[0] + d_ref[1] + 1.0
        q = lax.rsqrt(dp)
        t = jnp.dot(x_ref[...], w_ref[...], preferred_element_type=_f32)
        t_ref[...] = q * t
        q_ref[...] = q

    return pl.pallas_call(
        body,
        grid=(N // _BM,),
        in_specs=[
            pl.BlockSpec((_BM, 640), lambda i: (i, 0)),
            pl.BlockSpec((640, F), lambda i: (0, 0)),
            pl.BlockSpec((NC, _BM, F), lambda i: (0, i, 0)),
        ],
        out_specs=[
            pl.BlockSpec((_BM, F), lambda i: (i, 0)),
            pl.BlockSpec((_BM, F), lambda i: (i, 0)),
        ],
        out_shape=[
            jax.ShapeDtypeStruct((N, F), _f32),
            jax.ShapeDtypeStruct((N, F), _f32),
        ],
    )(x, W1, degp)


def _tc_mid1(zp, t1s, q16, b1, W2):
    def body(z_ref, t_ref, q_ref, b_ref, w_ref, o_ref):
        q = q_ref[...]
        z = z_ref[0] + z_ref[1] + t_ref[...]
        a = jnp.maximum(q * z + b_ref[...], 0.0)
        o_ref[...] = jnp.dot(q * a, w_ref[...], preferred_element_type=_f32)

    return pl.pallas_call(
        body,
        grid=(N // _BM,),
        in_specs=[
            pl.BlockSpec((NC, _BM, F), lambda i: (0, i, 0)),
            pl.BlockSpec((_BM, F), lambda i: (i, 0)),
            pl.BlockSpec((_BM, F), lambda i: (i, 0)),
            pl.BlockSpec((1, F), lambda i: (0, 0)),
            pl.BlockSpec((F, F), lambda i: (0, 0)),
        ],
        out_specs=pl.BlockSpec((_BM, F), lambda i: (i, 0)),
        out_shape=jax.ShapeDtypeStruct((N, F), _f32),
    )(zp, t1s, q16, b1, W2)


def _tc_mid2(zp, t2s, q16, b2):
    def body(z_ref, t_ref, q_ref, b_ref, o_ref):
        q = q_ref[...]
        z = z_ref[0] + z_ref[1] + t_ref[...]
        o_ref[...] = q * jnp.maximum(q * z + b_ref[...], 0.0)

    return pl.pallas_call(
        body,
        grid=(N // _BM,),
        in_specs=[
            pl.BlockSpec((NC, _BM, F), lambda i: (0, i, 0)),
            pl.BlockSpec((_BM, F), lambda i: (i, 0)),
            pl.BlockSpec((_BM, F), lambda i: (i, 0)),
            pl.BlockSpec((1, F), lambda i: (0, 0)),
        ],
        out_specs=pl.BlockSpec((_BM, F), lambda i: (i, 0)),
        out_shape=jax.ShapeDtypeStruct((N, F), _f32),
    )(zp, t2s, q16, b2)


def _tc_last(zp, s3, q16, W3, b3):
    def body(z_ref, s_ref, q_ref, w_ref, b_ref, o_ref):
        p = q_ref[...] * (z_ref[0] + z_ref[1] + s_ref[...])
        o_ref[...] = (
            jnp.dot(p, w_ref[...], preferred_element_type=_f32) + b_ref[...]
        )

    return pl.pallas_call(
        body,
        grid=(N // _BM,),
        in_specs=[
            pl.BlockSpec((NC, _BM, F), lambda i: (0, i, 0)),
            pl.BlockSpec((_BM, F), lambda i: (i, 0)),
            pl.BlockSpec((_BM, F), lambda i: (i, 0)),
            pl.BlockSpec((F, 640), lambda i: (0, 0)),
            pl.BlockSpec((1, 640), lambda i: (0, 0)),
        ],
        out_specs=pl.BlockSpec((_BM, 640), lambda i: (i, 0)),
        out_shape=jax.ShapeDtypeStruct((N, 640), _f32),
    )(zp, s3, q16, W3, b3)


def _pad_table(t):
    return jnp.pad(t, ((0, NP - N), (0, 0)))


def kernel(x, edges, W1, b1, W2, b2, W3, b3):
    src = edges[:, 0]
    dst = edges[:, 1]
    dummy = (N + (jnp.arange(EP - E, dtype=jnp.int32) % NS)).astype(jnp.int32)
    src_p = jnp.concatenate([src, dummy])
    dst_p = jnp.concatenate([dst, dummy])

    b1r = b1.reshape(1, F)
    b2r = b2.reshape(1, F)
    b3r = b3.reshape(1, 640)

    degp = _sc_degree(dst_p)[:, :N]
    t1s, q16 = _tc_first(x, W1, degp)

    z1p = _sc_prop(_pad_table(t1s), src_p, dst_p)[:, :N]
    t2s = _tc_mid1(z1p, t1s, q16, b1r, W2)

    z2p = _sc_prop(_pad_table(t2s), src_p, dst_p)[:, :N]
    s3 = _tc_mid2(z2p, t2s, q16, b2r)

    z3p = _sc_prop(_pad_table(s3), src_p, dst_p)[:, :N]
    out = _tc_last(z3p, s3, q16, W3, b3r)
    return out
